# Initial kernel scaffold; baseline (speedup 1.0000x reference)
import jax
import jax.numpy as jnp
from jax.experimental import pallas as pl


def kernel(pred_logits, pred_segments, pred_actionness, target_sizes):
    raise NotImplementedError("write your pallas kernel here")


# speedup vs baseline: 1.1144x; 1.1144x over previous
import functools

import jax
import jax.numpy as jnp
from jax.experimental import pallas as pl
from jax.experimental.pallas import tpu as pltpu

_K = 100
_LANES = 128
_BIGI = 2**30
_NEG = -1.0


def _body(logits_ref, aflat_ref, seg_ref, ts_ref,
          scores_ref, labels_ref, segout_ref, qid_ref,
          v_ref, lval_ref, lidx_ref, *, n_rows, n_rows_pad, n_classes):
    n_groups = n_rows_pad // 8

    v_ref[...] = jnp.full((n_rows_pad, _LANES), _NEG, jnp.float32)
    x = logits_ref[0]
    a = aflat_ref[0]
    v_ref[pl.ds(0, n_rows), :] = a / (1.0 + jnp.exp(-x))

    v3 = v_ref[...].reshape(n_groups, 8, _LANES)
    i3 = (
        (jax.lax.broadcasted_iota(jnp.int32, (n_groups, 8, _LANES), 0) * 8
         + jax.lax.broadcasted_iota(jnp.int32, (n_groups, 8, _LANES), 1)) * _LANES
        + jax.lax.broadcasted_iota(jnp.int32, (n_groups, 8, _LANES), 2)
    )
    gmax = jnp.max(v3, axis=1)
    gidx = jnp.min(jnp.where(v3 == gmax[:, None, :], i3, _BIGI), axis=1)
    lval_ref[...] = gmax
    lidx_ref[...] = gidx

    ts = ts_ref[pl.program_id(0), 0]
    base8 = (jax.lax.broadcasted_iota(jnp.int32, (8, _LANES), 0) * _LANES
             + jax.lax.broadcasted_iota(jnp.int32, (8, _LANES), 1))

    def step(j, carry):
        lv = lval_ref[...]
        li = lidx_ref[...]
        m = jnp.max(lv)
        istar = jnp.min(jnp.where(lv == m, li, _BIGI))
        rb = istar // (8 * _LANES)

        blk = v_ref[pl.ds(rb * 8, 8), :]
        fidx_blk = base8 + rb * (8 * _LANES)
        blk = jnp.where(fidx_blk == istar, _NEG, blk)
        v_ref[pl.ds(rb * 8, 8), :] = blk
        nmax = jnp.max(blk, axis=0, keepdims=True)
        nidx = jnp.min(jnp.where(blk == nmax, fidx_blk, _BIGI),
                       axis=0, keepdims=True)
        lval_ref[pl.ds(rb, 1), :] = nmax
        lidx_ref[pl.ds(rb, 1), :] = nidx

        q = istar // n_classes
        c = istar - q * n_classes
        scores_ref[0, pl.ds(j, 1), :] = m.reshape(1, 1)
        labels_ref[0, pl.ds(j, 1), :] = c.reshape(1, 1)
        qid_ref[0, pl.ds(j, 1), :] = q.reshape(1, 1)
        cw = seg_ref[0, pl.ds(q, 1), :]
        cc = cw[:, 0:1]
        ww = cw[:, 1:2]
        segout_ref[0, pl.ds(j, 1), :] = (
            jnp.concatenate([cc - 0.5 * ww, cc + 0.5 * ww], axis=1) * ts
        )
        return carry

    jax.lax.fori_loop(0, _K, step, 0)


def kernel(pred_logits, pred_segments, pred_actionness, target_sizes):
    B, Q, C = pred_logits.shape
    n = Q * C
    assert n % _LANES == 0
    n_rows = n // _LANES
    n_rows_pad = ((n_rows + 7) // 8) * 8
    n_groups = n_rows_pad // 8

    logits3 = pred_logits.reshape(B, n_rows, _LANES)
    aflat3 = jnp.repeat(pred_actionness[..., 0], C, axis=-1).reshape(
        B, n_rows, _LANES)
    ts2 = target_sizes.reshape(B, 1)

    grid = (B,)
    body = functools.partial(_body, n_rows=n_rows, n_rows_pad=n_rows_pad,
                             n_classes=C)
    scores, labels, segout, qids = pl.pallas_call(
        body,
        grid=grid,
        in_specs=[
            pl.BlockSpec((1, n_rows, _LANES), lambda b: (b, 0, 0)),
            pl.BlockSpec((1, n_rows, _LANES), lambda b: (b, 0, 0)),
            pl.BlockSpec((1, Q, 2), lambda b: (b, 0, 0)),
            pl.BlockSpec(memory_space=pltpu.SMEM),
        ],
        out_specs=[
            pl.BlockSpec((1, _K, 1), lambda b: (b, 0, 0)),
            pl.BlockSpec((1, _K, 1), lambda b: (b, 0, 0)),
            pl.BlockSpec((1, _K, 2), lambda b: (b, 0, 0)),
            pl.BlockSpec((1, _K, 1), lambda b: (b, 0, 0)),
        ],
        out_shape=[
            jax.ShapeDtypeStruct((B, _K, 1), jnp.float32),
            jax.ShapeDtypeStruct((B, _K, 1), jnp.int32),
            jax.ShapeDtypeStruct((B, _K, 2), jnp.float32),
            jax.ShapeDtypeStruct((B, _K, 1), jnp.int32),
        ],
        scratch_shapes=[
            pltpu.VMEM((n_rows_pad, _LANES), jnp.float32),
            pltpu.VMEM((n_groups, _LANES), jnp.float32),
            pltpu.VMEM((n_groups, _LANES), jnp.int32),
        ],
    )(logits3, aflat3, pred_segments, ts2)

    return (scores[..., 0], labels[..., 0], segout, qids[..., 0])


# speedup vs baseline: 1.1859x; 1.0642x over previous
import functools

import jax
import jax.numpy as jnp
from jax.experimental import pallas as pl
from jax.experimental.pallas import tpu as pltpu

_K = 100
_LANES = 128
_BIGI = 2**30
_NEG = -1.0


def _body(logits_ref, aflat_ref, seg_ref, ts_ref,
          scores_ref, labels_ref, segout_ref, qid_ref,
          v_ref, lval_ref, lidx_ref, l2val_ref, l2idx_ref,
          *, n_rows, n_rows_pad, n_classes):
    n_groups = n_rows_pad // 8
    n_groups_pad = ((n_groups + 7) // 8) * 8
    n_groups2 = n_groups_pad // 8

    v_ref[...] = jnp.full((n_rows_pad, _LANES), _NEG, jnp.float32)
    x = logits_ref[0]
    a = aflat_ref[0]
    v_ref[pl.ds(0, n_rows), :] = a / (1.0 + jnp.exp(-x))

    v3 = v_ref[...].reshape(n_groups, 8, _LANES)
    i3 = (
        (jax.lax.broadcasted_iota(jnp.int32, (n_groups, 8, _LANES), 0) * 8
         + jax.lax.broadcasted_iota(jnp.int32, (n_groups, 8, _LANES), 1)) * _LANES
        + jax.lax.broadcasted_iota(jnp.int32, (n_groups, 8, _LANES), 2)
    )
    gmax = jnp.max(v3, axis=1)
    gidx = jnp.min(jnp.where(v3 == gmax[:, None, :], i3, _BIGI), axis=1)
    lval_ref[...] = jnp.full((n_groups_pad, _LANES), _NEG, jnp.float32)
    lidx_ref[...] = jnp.full((n_groups_pad, _LANES), _BIGI, jnp.int32)
    lval_ref[pl.ds(0, n_groups), :] = gmax
    lidx_ref[pl.ds(0, n_groups), :] = gidx

    l1v3 = lval_ref[...].reshape(n_groups2, 8, _LANES)
    l1i3 = lidx_ref[...].reshape(n_groups2, 8, _LANES)
    g2max = jnp.max(l1v3, axis=1)
    g2idx = jnp.min(jnp.where(l1v3 == g2max[:, None, :], l1i3, _BIGI), axis=1)
    l2val_ref[...] = g2max
    l2idx_ref[...] = g2idx

    ts = ts_ref[pl.program_id(0), 0]
    base8 = (jax.lax.broadcasted_iota(jnp.int32, (8, _LANES), 0) * _LANES
             + jax.lax.broadcasted_iota(jnp.int32, (8, _LANES), 1))

    def step(j, carry):
        lv2 = l2val_ref[...]
        li2 = l2idx_ref[...]
        m = jnp.max(lv2)
        istar = jnp.min(jnp.where(lv2 == m, li2, _BIGI))
        rb = istar // (8 * _LANES)
        rb2 = rb // 8

        blk = v_ref[pl.ds(rb * 8, 8), :]
        fidx_blk = base8 + rb * (8 * _LANES)
        blk = jnp.where(fidx_blk == istar, _NEG, blk)
        v_ref[pl.ds(rb * 8, 8), :] = blk
        nmax = jnp.max(blk, axis=0, keepdims=True)
        nidx = jnp.min(jnp.where(blk == nmax, fidx_blk, _BIGI),
                       axis=0, keepdims=True)
        lval_ref[pl.ds(rb, 1), :] = nmax
        lidx_ref[pl.ds(rb, 1), :] = nidx

        l1blk = lval_ref[pl.ds(rb2 * 8, 8), :]
        l1iblk = lidx_ref[pl.ds(rb2 * 8, 8), :]
        nmax2 = jnp.max(l1blk, axis=0, keepdims=True)
        nidx2 = jnp.min(jnp.where(l1blk == nmax2, l1iblk, _BIGI),
                        axis=0, keepdims=True)
        l2val_ref[pl.ds(rb2, 1), :] = nmax2
        l2idx_ref[pl.ds(rb2, 1), :] = nidx2

        q = istar // n_classes
        c = istar - q * n_classes
        scores_ref[0, pl.ds(j, 1), :] = m.reshape(1, 1)
        labels_ref[0, pl.ds(j, 1), :] = c.reshape(1, 1)
        qid_ref[0, pl.ds(j, 1), :] = q.reshape(1, 1)
        cw = seg_ref[0, pl.ds(q, 1), :]
        cc = cw[:, 0:1]
        ww = cw[:, 1:2]
        segout_ref[0, pl.ds(j, 1), :] = (
            jnp.concatenate([cc - 0.5 * ww, cc + 0.5 * ww], axis=1) * ts
        )
        return carry

    jax.lax.fori_loop(0, _K, step, 0)


def kernel(pred_logits, pred_segments, pred_actionness, target_sizes):
    B, Q, C = pred_logits.shape
    n = Q * C
    assert n % _LANES == 0
    n_rows = n // _LANES
    n_rows_pad = ((n_rows + 7) // 8) * 8
    n_groups = n_rows_pad // 8
    n_groups_pad = ((n_groups + 7) // 8) * 8
    n_groups2 = n_groups_pad // 8

    logits3 = pred_logits.reshape(B, n_rows, _LANES)
    aflat3 = jnp.repeat(pred_actionness[..., 0], C, axis=-1).reshape(
        B, n_rows, _LANES)
    ts2 = target_sizes.reshape(B, 1)

    grid = (B,)
    body = functools.partial(_body, n_rows=n_rows, n_rows_pad=n_rows_pad,
                             n_classes=C)
    scores, labels, segout, qids = pl.pallas_call(
        body,
        grid=grid,
        in_specs=[
            pl.BlockSpec((1, n_rows, _LANES), lambda b: (b, 0, 0)),
            pl.BlockSpec((1, n_rows, _LANES), lambda b: (b, 0, 0)),
            pl.BlockSpec((1, Q, 2), lambda b: (b, 0, 0)),
            pl.BlockSpec(memory_space=pltpu.SMEM),
        ],
        out_specs=[
            pl.BlockSpec((1, _K, 1), lambda b: (b, 0, 0)),
            pl.BlockSpec((1, _K, 1), lambda b: (b, 0, 0)),
            pl.BlockSpec((1, _K, 2), lambda b: (b, 0, 0)),
            pl.BlockSpec((1, _K, 1), lambda b: (b, 0, 0)),
        ],
        out_shape=[
            jax.ShapeDtypeStruct((B, _K, 1), jnp.float32),
            jax.ShapeDtypeStruct((B, _K, 1), jnp.int32),
            jax.ShapeDtypeStruct((B, _K, 2), jnp.float32),
            jax.ShapeDtypeStruct((B, _K, 1), jnp.int32),
        ],
        scratch_shapes=[
            pltpu.VMEM((n_rows_pad, _LANES), jnp.float32),
            pltpu.VMEM((n_groups_pad, _LANES), jnp.float32),
            pltpu.VMEM((n_groups_pad, _LANES), jnp.int32),
            pltpu.VMEM((n_groups2, _LANES), jnp.float32),
            pltpu.VMEM((n_groups2, _LANES), jnp.int32),
        ],
    )(logits3, aflat3, pred_segments, ts2)

    return (scores[..., 0], labels[..., 0], segout, qids[..., 0])
